# Initial kernel scaffold; baseline (speedup 1.0000x reference)
import jax
import jax.numpy as jnp
from jax.experimental import pallas as pl


def kernel(x, edge_index, W1l, b1l, W1r, W2l, b2l, W2r, Wlin, blin):
    raise NotImplementedError("write your pallas kernel here")


# speedup vs baseline: 4.6856x; 4.6856x over previous
import jax
import jax.numpy as jnp
from jax import lax
from jax.experimental import pallas as pl
from jax.experimental.pallas import tpu as pltpu
from jax.experimental.pallas import tpu_sc as plsc

_N = 10000
_E = 320000
_D = 128

_NC = 2
_NS = 16
_NW = _NC * _NS
_EP = _E // _NW
_CH = 80
_NCH = _EP // _CH
_NP = 10240
_RP = _NP // _NS


def _sc_agg_body(x_hbm, src_hbm, dst_hbm, z128_hbm,
                 parts_out,
                 idx_s, idx_d, rows_v, acc_sh, sem_g, sem_a):
  cid = lax.axis_index("c")
  sid = lax.axis_index("s")
  wid = sid * _NC + cid
  r0 = sid * _RP
  pltpu.sync_copy(z128_hbm.at[pl.ds(r0, _RP)], acc_sh.at[pl.ds(r0, _RP)])
  plsc.subcore_barrier()
  ebase = wid * _EP

  @pl.loop(0, _NCH)
  def chunk(c):
    b = ebase + c * _CH
    pltpu.sync_copy(src_hbm.at[pl.ds(b, _CH)], idx_s)
    pltpu.async_copy(x_hbm.at[idx_s], rows_v, sem_g).wait()
    pltpu.sync_copy(dst_hbm.at[pl.ds(b, _CH)], idx_d)
    pltpu.async_copy(rows_v, acc_sh.at[idx_d], sem_a, add=True).wait()

  plsc.subcore_barrier()
  pltpu.sync_copy(acc_sh.at[pl.ds(r0, _RP)], parts_out.at[cid, pl.ds(r0, _RP)])


def _sc_cnt_body(dst_hbm, z128_hbm, ones_hbm,
                 cnt_out,
                 idx_d, ones_v, acc_sh, sem_a):
  cid = lax.axis_index("c")
  sid = lax.axis_index("s")
  wid = sid * _NC + cid
  r0 = sid * _RP
  pltpu.sync_copy(z128_hbm.at[pl.ds(r0, _RP)], acc_sh.at[pl.ds(r0, _RP)])
  pltpu.sync_copy(ones_hbm, ones_v)
  plsc.subcore_barrier()
  ebase = wid * _EP

  @pl.loop(0, _NCH)
  def chunk(c):
    b = ebase + c * _CH
    pltpu.sync_copy(dst_hbm.at[pl.ds(b, _CH)], idx_d)
    pltpu.async_copy(ones_v, acc_sh.at[idx_d], sem_a, add=True).wait()

  plsc.subcore_barrier()
  pltpu.sync_copy(acc_sh.at[pl.ds(r0, _RP)], cnt_out.at[cid, pl.ds(r0, _RP)])


def _make_sc_calls(interpret=False):
  mesh = plsc.VectorSubcoreMesh(core_axis_name="c", subcore_axis_name="s",
                                num_cores=_NC, num_subcores=_NS)
  agg = pl.kernel(
      _sc_agg_body,
      out_type=jax.ShapeDtypeStruct((_NC, _NP, _D), jnp.float32),
      mesh=mesh,
      scratch_types=[
          pltpu.VMEM((_CH,), jnp.int32),
          pltpu.VMEM((_CH,), jnp.int32),
          pltpu.VMEM((_CH, _D), jnp.float32),
          pltpu.VMEM_SHARED((_NP, _D), jnp.float32),
          pltpu.SemaphoreType.DMA,
          pltpu.SemaphoreType.DMA,
      ],
      interpret=interpret,
  )
  cnt = pl.kernel(
      _sc_cnt_body,
      out_type=jax.ShapeDtypeStruct((_NC, _NP, _D), jnp.float32),
      mesh=mesh,
      scratch_types=[
          pltpu.VMEM((_CH,), jnp.int32),
          pltpu.VMEM((_CH, _D), jnp.float32),
          pltpu.VMEM_SHARED((_NP, _D), jnp.float32),
          pltpu.SemaphoreType.DMA,
      ],
      interpret=interpret,
  )
  return agg, cnt


def _tc_layer1_body(parts_ref, cnt_ref, x_ref, wlt_ref, b_ref, wrt_ref, o_ref):
  s = parts_ref[0] + parts_ref[1]
  cnt = cnt_ref[0, :, 0:1] + cnt_ref[1, :, 0:1]
  agg = s / jnp.maximum(cnt, 1.0)
  h = (jnp.dot(agg, wlt_ref[...], preferred_element_type=jnp.float32)
       + b_ref[...]
       + jnp.dot(x_ref[...], wrt_ref[...], preferred_element_type=jnp.float32))
  o_ref[...] = jnp.maximum(h, 0.0)


def _tc_layer2_body(parts_ref, cnt_ref, h_ref, wlt_ref, b_ref, wrt_ref,
                    wlint_ref, blin_ref, o_ref):
  s = parts_ref[0] + parts_ref[1]
  cnt = cnt_ref[0, :, 0:1] + cnt_ref[1, :, 0:1]
  agg = s / jnp.maximum(cnt, 1.0)
  t = (jnp.dot(agg, wlt_ref[...], preferred_element_type=jnp.float32)
       + b_ref[...]
       + jnp.dot(h_ref[...], wrt_ref[...], preferred_element_type=jnp.float32))
  t = jnp.maximum(t, 0.0)
  o_ref[...] = (jnp.dot(t, wlint_ref[...], preferred_element_type=jnp.float32)
                + blin_ref[...])


_R = 400


def _tc_calls(interpret=False):
  grid = (_N // _R,)
  parts_spec = pl.BlockSpec((_NC, _R, _D), lambda i: (0, i, 0))
  row_spec = pl.BlockSpec((_R, _D), lambda i: (i, 0))
  w_spec = pl.BlockSpec((_D, _D), lambda i: (0, 0))
  b_spec = pl.BlockSpec((1, _D), lambda i: (0, 0))
  layer1 = pl.pallas_call(
      _tc_layer1_body,
      grid=grid,
      in_specs=[parts_spec, parts_spec, row_spec, w_spec, b_spec, w_spec],
      out_specs=row_spec,
      out_shape=jax.ShapeDtypeStruct((_N, _D), jnp.float32),
      interpret=interpret,
  )
  layer2 = pl.pallas_call(
      _tc_layer2_body,
      grid=grid,
      in_specs=[parts_spec, parts_spec, row_spec, w_spec, b_spec, w_spec,
                w_spec, b_spec],
      out_specs=row_spec,
      out_shape=jax.ShapeDtypeStruct((_N, _D), jnp.float32),
      interpret=interpret,
  )
  return layer1, layer2


@jax.jit
def kernel(x, edge_index, W1l, b1l, W1r, W2l, b2l, W2r, Wlin, blin):
  src = edge_index[0]
  dst = edge_index[1]
  z128 = jnp.zeros((_NP, _D), jnp.float32)
  ones = jnp.ones((_CH, _D), jnp.float32)

  sc_agg, sc_cnt = _make_sc_calls()
  tc1, tc2 = _tc_calls()

  cnt = sc_cnt(dst, z128, ones)
  parts1 = sc_agg(x, src, dst, z128)
  h = tc1(parts1, cnt, x, W1l.T, b1l.reshape(1, _D), W1r.T)
  parts2 = sc_agg(h, src, dst, z128)
  out = tc2(parts2, cnt, h, W2l.T, b2l.reshape(1, _D), W2r.T,
            Wlin.T, blin.reshape(1, _D))
  return out


# speedup vs baseline: 5.3566x; 1.1432x over previous
import jax
import jax.numpy as jnp
from jax import lax
from jax.experimental import pallas as pl
from jax.experimental.pallas import tpu as pltpu
from jax.experimental.pallas import tpu_sc as plsc

_N = 10000
_E = 320000
_D = 128

_NC = 2
_NS = 16
_NW = _NC * _NS
_EP = _E // _NW
_CH = 80
_NCH = _EP // _CH
_NP = 10240
_RP = _NP // _NS


def _sc_agg_body(x_hbm, src_hbm, dst_hbm, z128_hbm,
                 parts_out,
                 idx_s0, idx_s1, idx_d0, idx_d1, rows0, rows1, acc_sh,
                 gsem0, gsem1, asem0, asem1):
  cid = lax.axis_index("c")
  sid = lax.axis_index("s")
  wid = sid * _NC + cid
  r0 = sid * _RP
  pltpu.sync_copy(z128_hbm.at[pl.ds(r0, _RP)], acc_sh.at[pl.ds(r0, _RP)])
  plsc.subcore_barrier()
  ebase = wid * _EP

  idx_s = (idx_s0, idx_s1)
  idx_d = (idx_d0, idx_d1)
  rows = (rows0, rows1)
  gsem = (gsem0, gsem1)
  asem = (asem0, asem1)

  pltpu.sync_copy(src_hbm.at[pl.ds(ebase, _CH)], idx_s[0])
  pltpu.async_copy(x_hbm.at[idx_s[0]], rows[0], gsem[0])

  @pl.loop(0, _NCH + 1, step=2)
  def grp(g):
    for b in (0, 1):
      c = g + b
      o = 1 - b

      @pl.when(c < _NCH)
      def _():
        pltpu.make_async_copy(x_hbm.at[idx_s[b]], rows[b], gsem[b]).wait()
        pltpu.sync_copy(dst_hbm.at[pl.ds(ebase + c * _CH, _CH)], idx_d[b])

        @pl.when(c > 0)
        def _():
          pltpu.make_async_copy(rows[o], acc_sh.at[idx_d[o]], asem[o]).wait()

        pltpu.async_copy(rows[b], acc_sh.at[idx_d[b]], asem[b], add=True)

        @pl.when(c + 1 < _NCH)
        def _():
          pltpu.sync_copy(src_hbm.at[pl.ds(ebase + (c + 1) * _CH, _CH)],
                          idx_s[o])
          pltpu.async_copy(x_hbm.at[idx_s[o]], rows[o], gsem[o])

  bl = (_NCH - 1) % 2
  pltpu.make_async_copy(rows[bl], acc_sh.at[idx_d[bl]], asem[bl]).wait()
  plsc.subcore_barrier()
  pltpu.sync_copy(acc_sh.at[pl.ds(r0, _RP)], parts_out.at[cid, pl.ds(r0, _RP)])


def _sc_cnt_body(dst_hbm, z128_hbm, ones_hbm,
                 cnt_out,
                 idx_d, ones_v, acc_sh, sem_a):
  cid = lax.axis_index("c")
  sid = lax.axis_index("s")
  wid = sid * _NC + cid
  r0 = sid * _RP
  pltpu.sync_copy(z128_hbm.at[pl.ds(r0, _RP)], acc_sh.at[pl.ds(r0, _RP)])
  pltpu.sync_copy(ones_hbm, ones_v)
  plsc.subcore_barrier()
  ebase = wid * _EP

  @pl.loop(0, _NCH)
  def chunk(c):
    b = ebase + c * _CH
    pltpu.sync_copy(dst_hbm.at[pl.ds(b, _CH)], idx_d)
    pltpu.async_copy(ones_v, acc_sh.at[idx_d], sem_a, add=True).wait()

  plsc.subcore_barrier()
  pltpu.sync_copy(acc_sh.at[pl.ds(r0, _RP)], cnt_out.at[cid, pl.ds(r0, _RP)])


def _make_sc_calls(interpret=False):
  mesh = plsc.VectorSubcoreMesh(core_axis_name="c", subcore_axis_name="s",
                                num_cores=_NC, num_subcores=_NS)
  agg = pl.kernel(
      _sc_agg_body,
      out_type=jax.ShapeDtypeStruct((_NC, _NP, _D), jnp.float32),
      mesh=mesh,
      scratch_types=[
          pltpu.VMEM((_CH,), jnp.int32),
          pltpu.VMEM((_CH,), jnp.int32),
          pltpu.VMEM((_CH,), jnp.int32),
          pltpu.VMEM((_CH,), jnp.int32),
          pltpu.VMEM((_CH, _D), jnp.float32),
          pltpu.VMEM((_CH, _D), jnp.float32),
          pltpu.VMEM_SHARED((_NP, _D), jnp.float32),
          pltpu.SemaphoreType.DMA,
          pltpu.SemaphoreType.DMA,
          pltpu.SemaphoreType.DMA,
          pltpu.SemaphoreType.DMA,
      ],
      interpret=interpret,
  )
  cnt = pl.kernel(
      _sc_cnt_body,
      out_type=jax.ShapeDtypeStruct((_NC, _NP, _D), jnp.float32),
      mesh=mesh,
      scratch_types=[
          pltpu.VMEM((_CH,), jnp.int32),
          pltpu.VMEM((_CH, _D), jnp.float32),
          pltpu.VMEM_SHARED((_NP, _D), jnp.float32),
          pltpu.SemaphoreType.DMA,
      ],
      interpret=interpret,
  )
  return agg, cnt


def _tc_layer1_body(parts_ref, cnt_ref, x_ref, wlt_ref, b_ref, wrt_ref, o_ref):
  s = parts_ref[0] + parts_ref[1]
  cnt = cnt_ref[0, :, 0:1] + cnt_ref[1, :, 0:1]
  agg = s / jnp.maximum(cnt, 1.0)
  h = (jnp.dot(agg, wlt_ref[...], preferred_element_type=jnp.float32)
       + b_ref[...]
       + jnp.dot(x_ref[...], wrt_ref[...], preferred_element_type=jnp.float32))
  o_ref[...] = jnp.maximum(h, 0.0)


def _tc_layer2_body(parts_ref, cnt_ref, h_ref, wlt_ref, b_ref, wrt_ref,
                    wlint_ref, blin_ref, o_ref):
  s = parts_ref[0] + parts_ref[1]
  cnt = cnt_ref[0, :, 0:1] + cnt_ref[1, :, 0:1]
  agg = s / jnp.maximum(cnt, 1.0)
  t = (jnp.dot(agg, wlt_ref[...], preferred_element_type=jnp.float32)
       + b_ref[...]
       + jnp.dot(h_ref[...], wrt_ref[...], preferred_element_type=jnp.float32))
  t = jnp.maximum(t, 0.0)
  o_ref[...] = (jnp.dot(t, wlint_ref[...], preferred_element_type=jnp.float32)
                + blin_ref[...])


_R = 400


def _tc_calls(interpret=False):
  grid = (_N // _R,)
  parts_spec = pl.BlockSpec((_NC, _R, _D), lambda i: (0, i, 0))
  row_spec = pl.BlockSpec((_R, _D), lambda i: (i, 0))
  w_spec = pl.BlockSpec((_D, _D), lambda i: (0, 0))
  b_spec = pl.BlockSpec((1, _D), lambda i: (0, 0))
  layer1 = pl.pallas_call(
      _tc_layer1_body,
      grid=grid,
      in_specs=[parts_spec, parts_spec, row_spec, w_spec, b_spec, w_spec],
      out_specs=row_spec,
      out_shape=jax.ShapeDtypeStruct((_N, _D), jnp.float32),
      interpret=interpret,
  )
  layer2 = pl.pallas_call(
      _tc_layer2_body,
      grid=grid,
      in_specs=[parts_spec, parts_spec, row_spec, w_spec, b_spec, w_spec,
                w_spec, b_spec],
      out_specs=row_spec,
      out_shape=jax.ShapeDtypeStruct((_N, _D), jnp.float32),
      interpret=interpret,
  )
  return layer1, layer2


@jax.jit
def kernel(x, edge_index, W1l, b1l, W1r, W2l, b2l, W2r, Wlin, blin):
  src = edge_index[0]
  dst = edge_index[1]
  z128 = jnp.zeros((_NP, _D), jnp.float32)
  ones = jnp.ones((_CH, _D), jnp.float32)

  sc_agg, sc_cnt = _make_sc_calls()
  tc1, tc2 = _tc_calls()

  cnt = sc_cnt(dst, z128, ones)
  parts1 = sc_agg(x, src, dst, z128)
  h = tc1(parts1, cnt, x, W1l.T, b1l.reshape(1, _D), W1r.T)
  parts2 = sc_agg(h, src, dst, z128)
  out = tc2(parts2, cnt, h, W2l.T, b2l.reshape(1, _D), W2r.T,
            Wlin.T, blin.reshape(1, _D))
  return out


# speedup vs baseline: 6.4746x; 1.2087x over previous
import jax
import jax.numpy as jnp
from jax import lax
from jax.experimental import pallas as pl
from jax.experimental.pallas import tpu as pltpu
from jax.experimental.pallas import tpu_sc as plsc

_N = 10000
_E = 320000
_D = 128

_NC = 2
_NS = 16
_NW = _NC * _NS
_EP = _E // _NW
_CH = 80
_NCH = _EP // _CH
_NP = 10240
_RP = _NP // _NS


def _sc_agg_body(x_hbm, ei_hbm, z128_hbm,
                 parts_out,
                 ei0, ei1, rows0, rows1, acc_sh,
                 gsem0, gsem1, asem0, asem1):
  cid = lax.axis_index("c")
  sid = lax.axis_index("s")
  wid = sid * _NC + cid
  r0 = sid * _RP
  pltpu.sync_copy(z128_hbm.at[pl.ds(r0, _RP)], acc_sh.at[pl.ds(r0, _RP)])
  plsc.subcore_barrier()
  cbase = wid * _NCH

  ei = (ei0, ei1)
  rows = (rows0, rows1)
  gsem = (gsem0, gsem1)
  asem = (asem0, asem1)

  pltpu.sync_copy(ei_hbm.at[cbase], ei[0])
  pltpu.async_copy(x_hbm.at[ei[0].at[0]], rows[0], gsem[0])

  @pl.loop(0, _NCH + 1, step=2)
  def grp(g):
    for b in (0, 1):
      c = g + b
      o = 1 - b

      @pl.when(c < _NCH)
      def _():
        pltpu.make_async_copy(x_hbm.at[ei[b].at[0]], rows[b], gsem[b]).wait()

        @pl.when(c > 0)
        def _():
          pltpu.make_async_copy(rows[o], acc_sh.at[ei[o].at[1]], asem[o]).wait()

        pltpu.async_copy(rows[b], acc_sh.at[ei[b].at[1]], asem[b], add=True)

        @pl.when(c + 1 < _NCH)
        def _():
          pltpu.sync_copy(ei_hbm.at[cbase + c + 1], ei[o])
          pltpu.async_copy(x_hbm.at[ei[o].at[0]], rows[o], gsem[o])

  bl = (_NCH - 1) % 2
  pltpu.make_async_copy(rows[bl], acc_sh.at[ei[bl].at[1]], asem[bl]).wait()
  plsc.subcore_barrier()
  pltpu.sync_copy(acc_sh.at[pl.ds(r0, _RP)], parts_out.at[cid, pl.ds(r0, _RP)])


def _sc_cnt_body(ei_hbm, z128_hbm, ones_hbm,
                 cnt_out,
                 ei0, ei1, ones_v, acc_sh, sem_a0, sem_a1):
  cid = lax.axis_index("c")
  sid = lax.axis_index("s")
  wid = sid * _NC + cid
  r0 = sid * _RP
  pltpu.sync_copy(z128_hbm.at[pl.ds(r0, _RP)], acc_sh.at[pl.ds(r0, _RP)])
  pltpu.sync_copy(ones_hbm, ones_v)
  plsc.subcore_barrier()
  cbase = wid * _NCH
  ei = (ei0, ei1)
  asem = (sem_a0, sem_a1)
  pltpu.sync_copy(ei_hbm.at[cbase], ei[0])

  @pl.loop(0, _NCH + 1, step=2)
  def grp(g):
    for b in (0, 1):
      c = g + b
      o = 1 - b

      @pl.when(c < _NCH)
      def _():
        @pl.when(c > 0)
        def _():
          pltpu.make_async_copy(ones_v, acc_sh.at[ei[o].at[1]], asem[o]).wait()

        pltpu.async_copy(ones_v, acc_sh.at[ei[b].at[1]], asem[b], add=True)

        @pl.when(c + 1 < _NCH)
        def _():
          pltpu.sync_copy(ei_hbm.at[cbase + c + 1], ei[o])

  bl = (_NCH - 1) % 2
  pltpu.make_async_copy(ones_v, acc_sh.at[ei[bl].at[1]], asem[bl]).wait()
  plsc.subcore_barrier()
  pltpu.sync_copy(acc_sh.at[pl.ds(r0, _RP)], cnt_out.at[cid, pl.ds(r0, _RP)])


def _make_sc_calls(interpret=False):
  mesh = plsc.VectorSubcoreMesh(core_axis_name="c", subcore_axis_name="s",
                                num_cores=_NC, num_subcores=_NS)
  agg = pl.kernel(
      _sc_agg_body,
      out_type=jax.ShapeDtypeStruct((_NC, _NP, _D), jnp.float32),
      mesh=mesh,
      scratch_types=[
          pltpu.VMEM((2, _CH), jnp.int32),
          pltpu.VMEM((2, _CH), jnp.int32),
          pltpu.VMEM((_CH, _D), jnp.float32),
          pltpu.VMEM((_CH, _D), jnp.float32),
          pltpu.VMEM_SHARED((_NP, _D), jnp.float32),
          pltpu.SemaphoreType.DMA,
          pltpu.SemaphoreType.DMA,
          pltpu.SemaphoreType.DMA,
          pltpu.SemaphoreType.DMA,
      ],
      interpret=interpret,
  )
  cnt = pl.kernel(
      _sc_cnt_body,
      out_type=jax.ShapeDtypeStruct((_NC, _NP, _D), jnp.float32),
      mesh=mesh,
      scratch_types=[
          pltpu.VMEM((2, _CH), jnp.int32),
          pltpu.VMEM((2, _CH), jnp.int32),
          pltpu.VMEM((_CH, _D), jnp.float32),
          pltpu.VMEM_SHARED((_NP, _D), jnp.float32),
          pltpu.SemaphoreType.DMA,
          pltpu.SemaphoreType.DMA,
      ],
      interpret=interpret,
  )
  return agg, cnt


def _tc_layer1_body(parts_ref, cnt_ref, x_ref, wlt_ref, b_ref, wrt_ref, o_ref):
  s = parts_ref[0] + parts_ref[1]
  cnt = cnt_ref[0, :, 0:1] + cnt_ref[1, :, 0:1]
  agg = s / jnp.maximum(cnt, 1.0)
  h = (jnp.dot(agg, wlt_ref[...], preferred_element_type=jnp.float32)
       + b_ref[...]
       + jnp.dot(x_ref[...], wrt_ref[...], preferred_element_type=jnp.float32))
  o_ref[...] = jnp.maximum(h, 0.0)


def _tc_layer2_body(parts_ref, cnt_ref, h_ref, wlt_ref, b_ref, wrt_ref,
                    wlint_ref, blin_ref, o_ref):
  s = parts_ref[0] + parts_ref[1]
  cnt = cnt_ref[0, :, 0:1] + cnt_ref[1, :, 0:1]
  agg = s / jnp.maximum(cnt, 1.0)
  t = (jnp.dot(agg, wlt_ref[...], preferred_element_type=jnp.float32)
       + b_ref[...]
       + jnp.dot(h_ref[...], wrt_ref[...], preferred_element_type=jnp.float32))
  t = jnp.maximum(t, 0.0)
  o_ref[...] = (jnp.dot(t, wlint_ref[...], preferred_element_type=jnp.float32)
                + blin_ref[...])


_R = 400


def _tc_calls(interpret=False):
  grid = (_N // _R,)
  parts_spec = pl.BlockSpec((_NC, _R, _D), lambda i: (0, i, 0))
  row_spec = pl.BlockSpec((_R, _D), lambda i: (i, 0))
  w_spec = pl.BlockSpec((_D, _D), lambda i: (0, 0))
  b_spec = pl.BlockSpec((1, _D), lambda i: (0, 0))
  layer1 = pl.pallas_call(
      _tc_layer1_body,
      grid=grid,
      in_specs=[parts_spec, parts_spec, row_spec, w_spec, b_spec, w_spec],
      out_specs=row_spec,
      out_shape=jax.ShapeDtypeStruct((_N, _D), jnp.float32),
      interpret=interpret,
  )
  layer2 = pl.pallas_call(
      _tc_layer2_body,
      grid=grid,
      in_specs=[parts_spec, parts_spec, row_spec, w_spec, b_spec, w_spec,
                w_spec, b_spec],
      out_specs=row_spec,
      out_shape=jax.ShapeDtypeStruct((_N, _D), jnp.float32),
      interpret=interpret,
  )
  return layer1, layer2


@jax.jit
def kernel(x, edge_index, W1l, b1l, W1r, W2l, b2l, W2r, Wlin, blin):
  ei = jnp.stack([edge_index[0].reshape(_NW, _NCH, _CH),
                  edge_index[1].reshape(_NW, _NCH, _CH)],
                 axis=2).reshape(_NW * _NCH, 2, _CH)
  z128 = jnp.zeros((_NP, _D), jnp.float32)
  ones = jnp.ones((_CH, _D), jnp.float32)

  sc_agg, sc_cnt = _make_sc_calls()
  tc1, tc2 = _tc_calls()

  cnt = sc_cnt(ei, z128, ones)
  parts1 = sc_agg(x, ei, z128)
  h = tc1(parts1, cnt, x, W1l.T, b1l.reshape(1, _D), W1r.T)
  parts2 = sc_agg(h, ei, z128)
  out = tc2(parts2, cnt, h, W2l.T, b2l.reshape(1, _D), W2r.T,
            Wlin.T, blin.reshape(1, _D))
  return out
